# baseline (speedup 1.0000x reference)
import functools

import jax
import jax.numpy as jnp
from jax import lax
from jax.experimental import pallas as pl
from jax.experimental.pallas import tpu as pltpu
from jax.experimental.pallas import tpu_sc as plsc

NC = 2
NS = 16
L = 16
CHUNK = 128


@functools.cache
def _make_glove_kernel(V: int, D: int, B: int):
    NW = NC * NS
    bpw = B // NW
    n_chunks = bpw // CHUNK
    n_groups = bpw // L

    mesh = plsc.VectorSubcoreMesh(core_axis_name="c", subcore_axis_name="s")

    @functools.partial(
        pl.kernel,
        out_type=jax.ShapeDtypeStruct((B,), jnp.float32),
        mesh=mesh,
        compiler_params=pltpu.CompilerParams(
            needs_layout_passes=False, use_tc_tiling_on_sc=False),
        scratch_types=[
            pltpu.VMEM((n_chunks, CHUNK), jnp.int32),
            pltpu.VMEM((n_chunks, CHUNK), jnp.int32),
            pltpu.VMEM((bpw, D), jnp.float32),
            pltpu.VMEM((bpw, D), jnp.float32),
            pltpu.VMEM((bpw,), jnp.float32),
            pltpu.VMEM((bpw,), jnp.float32),
            pltpu.VMEM((bpw,), jnp.float32),
            pltpu.SemaphoreType.DMA,
        ],
    )
    def glove(i_hbm, j_hbm, wi_hbm, wj_hbm, bi_hbm, bj_hbm, out_hbm,
              idx_i, idx_j, rows_i, rows_j, bias_i, bias_j, out_v, sem):
        wid = lax.axis_index("s") * NC + lax.axis_index("c")
        base = wid * bpw

        pltpu.sync_copy(i_hbm.at[pl.ds(wid * n_chunks, n_chunks)], idx_i)
        pltpu.sync_copy(j_hbm.at[pl.ds(wid * n_chunks, n_chunks)], idx_j)

        copies = []
        for k in range(n_chunks):
            rsl = pl.ds(k * CHUNK, CHUNK)
            copies.append(pltpu.async_copy(
                wi_hbm.at[idx_i.at[k]], rows_i.at[rsl], sem))
            copies.append(pltpu.async_copy(
                wj_hbm.at[idx_j.at[k]], rows_j.at[rsl], sem))
            copies.append(pltpu.async_copy(
                bi_hbm.at[idx_i.at[k]], bias_i.at[rsl], sem))
            copies.append(pltpu.async_copy(
                bj_hbm.at[idx_j.at[k]], bias_j.at[rsl], sem))
        for c in copies:
            c.wait()

        lane = lax.iota(jnp.int32, L)

        def group_body(g, carry):
            rid = lane + g * L
            acc = [jnp.zeros((L,), jnp.float32) for _ in range(4)]
            for d in range(D):
                dvec = jnp.full((L,), d, jnp.int32)
                acc[d % 4] = acc[d % 4] + (
                    plsc.load_gather(rows_i, [rid, dvec])
                    * plsc.load_gather(rows_j, [rid, dvec]))
            tot = (acc[0] + acc[1]) + (acc[2] + acc[3])
            osl = pl.ds(g * L, L)
            out_v[osl] = tot + bias_i[osl] + bias_j[osl]
            return carry

        lax.fori_loop(0, n_groups, group_body, 0)
        pltpu.sync_copy(out_v, out_hbm.at[pl.ds(base, bpw)])

    return glove


def kernel(i_indices, j_indices, wi, wj, bi, bj):
    V, D = wi.shape
    B = i_indices.shape[0]
    ii = i_indices.astype(jnp.int32).reshape(B // CHUNK, CHUNK)
    jj = j_indices.astype(jnp.int32).reshape(B // CHUNK, CHUNK)
    glove = _make_glove_kernel(V, D, B)
    return glove(ii, jj, wi, wj, bi[:, 0], bj[:, 0])
